# baseline (device time: 132139 ns/iter reference)
import jax
import jax.numpy as jnp
from jax import lax
from jax.experimental import pallas as pl
from jax.experimental.pallas import tpu as pltpu

N_DEV = 4
B = 2
SQ = 512
DM = 768
SKV_SH = 512
H_SH = 8
DH = 64
HD_SH = H_SH * DH
BLK = 64
NRES = 4

bf16 = jnp.bfloat16
f32 = jnp.float32


def _pack(K_ext, V_ext):
    def body(k_ref, v_ref, kp_ref, vp_ref):
        for d in range(N_DEV):
            for r in range(NRES):
                for i, blk in enumerate((r, r + 4)):
                    rows = slice(BLK * blk, BLK * (blk + 1))
                    cols = slice(HD_SH * d, HD_SH * (d + 1))
                    dst = slice(BLK * i, BLK * (i + 1))
                    kp_ref[d, r, :, dst] = k_ref[:, rows, cols].astype(bf16)
                    vp_ref[d, r, :, dst] = v_ref[:, rows, cols].astype(bf16)

    shp = jax.ShapeDtypeStruct((N_DEV, NRES, B, 2 * BLK, HD_SH), bf16)
    return pl.pallas_call(
        body,
        out_shape=(shp, shp),
        in_specs=[pl.BlockSpec(memory_space=pltpu.VMEM)] * 2,
        out_specs=(pl.BlockSpec(memory_space=pltpu.VMEM),) * 2,
        compiler_params=pltpu.CompilerParams(
            vmem_limit_bytes=64 * 1024 * 1024,
        ),
    )(K_ext.reshape(B, SKV_SH, N_DEV * HD_SH),
      V_ext.reshape(B, SKV_SH, N_DEV * HD_SH))


def kernel(x, Wq, K_ext, V_ext, Wo):
    kpk, vpk = _pack(K_ext, V_ext)

    def body(x_ref, wq_ref, kpk_ref, vpk_ref, wo_ref, out_ref,
             kgath, vgath, cbuf, pbuf, rsbuf, qred, agbuf,
             ksend_sems, vsend_sems, krecv_sems, vrecv_sems,
             rssend_sems, rsrecv_sems, agsend_sems, agrecv_sems):
        my = lax.axis_index("i")

        for m in range(N_DEV):
            for r in range(NRES):
                for d in range(N_DEV):
                    if d == m:
                        continue

                    @pl.when(my == m)
                    def _(m=m, d=d, r=r):
                        for src, gath, ssem, rsem in (
                            (kpk_ref, kgath, ksend_sems, krecv_sems),
                            (vpk_ref, vgath, vsend_sems, vrecv_sems),
                        ):
                            rdma = pltpu.make_async_remote_copy(
                                src_ref=src.at[d, r],
                                dst_ref=gath.at[m, r],
                                send_sem=ssem.at[d, r],
                                recv_sem=rsem.at[m, r],
                                device_id=(d,),
                                device_id_type=pl.DeviceIdType.MESH,
                            )
                            rdma.start()

        for m in range(N_DEV):
            @pl.when(my == m)
            def _(m=m):
                for r in range(NRES):
                    kgath[m, r] = kpk_ref[m, r]
                    vgath[m, r] = vpk_ref[m, r]

        wq = wq_ref[...].astype(bf16)
        qs = []
        for b in range(B):
            qf = jnp.dot(x_ref[b].astype(bf16), wq,
                         preferred_element_type=f32)
            qs.append((qf * 0.125).astype(bf16))

        for r in range(NRES):
            for s in range(N_DEV):
                @pl.when(s != my)
                def _(s=s, r=r):
                    for src, gath, ssem, rsem in (
                        (kpk_ref, kgath, ksend_sems, krecv_sems),
                        (vpk_ref, vgath, vsend_sems, vrecv_sems),
                    ):
                        rdma = pltpu.make_async_remote_copy(
                            src_ref=src.at[s, r],
                            dst_ref=gath.at[s, r],
                            send_sem=ssem.at[s, r],
                            recv_sem=rsem.at[s, r],
                            device_id=(s,),
                            device_id_type=pl.DeviceIdType.MESH,
                        )
                        rdma.wait_recv()

            for b in range(B):
                q_b = qs[b]
                qr = jnp.concatenate(
                    [q_b[BLK * r:BLK * (r + 1)],
                     q_b[BLK * (r + 4):BLK * (r + 5)]], axis=0)
                kr = kgath[:, r, b].reshape(8 * BLK, HD_SH)
                vr = vgath[:, r, b].reshape(8 * BLK, HD_SH)
                qr3 = qr.reshape(2 * BLK, H_SH, DH)
                kr3 = kr.reshape(8 * BLK, H_SH, DH)
                vr3 = vr.reshape(8 * BLK, H_SH, DH)
                scores = lax.dot_general(
                    qr3, kr3, (((2,), (2,)), ((1,), (1,))),
                    preferred_element_type=f32)
                mx = jnp.max(scores, axis=-1, keepdims=True)
                w = jnp.exp(scores - mx)
                w = w / jnp.sum(w, axis=-1, keepdims=True)
                ctx = lax.dot_general(
                    w.astype(bf16), vr3, (((2,), (0,)), ((0,), (1,))),
                    preferred_element_type=f32)
                ctx = jnp.swapaxes(ctx, 0, 1).reshape(2 * BLK, HD_SH)
                ctx = ctx.astype(bf16)
                cbuf[b, BLK * r:BLK * (r + 1)] = ctx[:BLK]
                cbuf[b, BLK * (r + 4):BLK * (r + 5)] = ctx[BLK:]

        for d in range(N_DEV):
            @pl.when(d != my)
            def _(d=d):
                for r in range(NRES):
                    for src, gath, ssem, rsem in (
                        (kpk_ref, kgath, ksend_sems, krecv_sems),
                        (vpk_ref, vgath, vsend_sems, vrecv_sems),
                    ):
                        rdma = pltpu.make_async_remote_copy(
                            src_ref=src.at[d, r],
                            dst_ref=gath.at[d, r],
                            send_sem=ssem.at[d, r],
                            recv_sem=rsem.at[d, r],
                            device_id=(d,),
                            device_id_type=pl.DeviceIdType.MESH,
                        )
                        rdma.wait_send()

        wo = wo_ref[...].astype(bf16)
        for b in range(B):
            pbuf[b] = jnp.dot(cbuf[b], wo,
                              preferred_element_type=f32).astype(bf16)

        QR = SQ // N_DEV
        for m in range(N_DEV):
            for d in range(N_DEV):
                if d == m:
                    continue

                @pl.when(my == m)
                def _(m=m, d=d):
                    rdma = pltpu.make_async_remote_copy(
                        src_ref=pbuf.at[:, QR * d:QR * (d + 1), :],
                        dst_ref=rsbuf.at[m],
                        send_sem=rssend_sems.at[d],
                        recv_sem=rsrecv_sems.at[m],
                        device_id=(d,),
                        device_id_type=pl.DeviceIdType.MESH,
                    )
                    rdma.start()

        for s in range(N_DEV):
            @pl.when(s != my)
            def _(s=s):
                rdma = pltpu.make_async_remote_copy(
                    src_ref=pbuf.at[:, QR * s:QR * (s + 1), :],
                    dst_ref=rsbuf.at[s],
                    send_sem=rssend_sems.at[s],
                    recv_sem=rsrecv_sems.at[s],
                    device_id=(s,),
                    device_id_type=pl.DeviceIdType.MESH,
                )
                rdma.wait_recv()

        for m in range(N_DEV):
            @pl.when(my == m)
            def _(m=m):
                acc = pbuf[:, QR * m:QR * (m + 1), :].astype(f32)
                for s in range(N_DEV):
                    if s != m:
                        acc = acc + rsbuf[s].astype(f32)
                out_ref[:, QR * m:QR * (m + 1), :] = acc
                qred[...] = acc.astype(bf16)

        for m in range(N_DEV):
            for d in range(N_DEV):
                if d == m:
                    continue

                @pl.when(my == m)
                def _(m=m, d=d):
                    rdma = pltpu.make_async_remote_copy(
                        src_ref=qred,
                        dst_ref=agbuf.at[m],
                        send_sem=agsend_sems.at[d],
                        recv_sem=agrecv_sems.at[m],
                        device_id=(d,),
                        device_id_type=pl.DeviceIdType.MESH,
                    )
                    rdma.start()

        for s in range(N_DEV):
            @pl.when(s != my)
            def _(s=s):
                rdma = pltpu.make_async_remote_copy(
                    src_ref=qred,
                    dst_ref=agbuf.at[s],
                    send_sem=agsend_sems.at[s],
                    recv_sem=agrecv_sems.at[s],
                    device_id=(s,),
                    device_id_type=pl.DeviceIdType.MESH,
                )
                rdma.wait_recv()
                out_ref[:, QR * s:QR * (s + 1), :] = agbuf[s].astype(f32)

        for d in range(N_DEV):
            @pl.when(d != my)
            def _(d=d):
                rs = pltpu.make_async_remote_copy(
                    src_ref=pbuf.at[:, QR * d:QR * (d + 1), :],
                    dst_ref=rsbuf.at[d],
                    send_sem=rssend_sems.at[d],
                    recv_sem=rsrecv_sems.at[d],
                    device_id=(d,),
                    device_id_type=pl.DeviceIdType.MESH,
                )
                rs.wait_send()
                ag = pltpu.make_async_remote_copy(
                    src_ref=qred,
                    dst_ref=agbuf.at[d],
                    send_sem=agsend_sems.at[d],
                    recv_sem=agrecv_sems.at[d],
                    device_id=(d,),
                    device_id_type=pl.DeviceIdType.MESH,
                )
                ag.wait_send()

    out_shape = jax.ShapeDtypeStruct((B, SQ, DM), f32)
    return pl.pallas_call(
        body,
        out_shape=out_shape,
        in_specs=[pl.BlockSpec(memory_space=pltpu.VMEM)] * 5,
        out_specs=pl.BlockSpec(memory_space=pltpu.VMEM),
        scratch_shapes=[
            pltpu.VMEM((N_DEV, NRES, B, 2 * BLK, HD_SH), bf16),
            pltpu.VMEM((N_DEV, NRES, B, 2 * BLK, HD_SH), bf16),
            pltpu.VMEM((B, SQ, HD_SH), bf16),
            pltpu.VMEM((B, SQ, DM), bf16),
            pltpu.VMEM((N_DEV, B, SQ // N_DEV, DM), bf16),
            pltpu.VMEM((B, SQ // N_DEV, DM), bf16),
            pltpu.VMEM((N_DEV, B, SQ // N_DEV, DM), bf16),
            pltpu.SemaphoreType.DMA((N_DEV, NRES)),
            pltpu.SemaphoreType.DMA((N_DEV, NRES)),
            pltpu.SemaphoreType.DMA((N_DEV, NRES)),
            pltpu.SemaphoreType.DMA((N_DEV, NRES)),
            pltpu.SemaphoreType.DMA((N_DEV,)),
            pltpu.SemaphoreType.DMA((N_DEV,)),
            pltpu.SemaphoreType.DMA((N_DEV,)),
            pltpu.SemaphoreType.DMA((N_DEV,)),
        ],
        compiler_params=pltpu.CompilerParams(
            vmem_limit_bytes=64 * 1024 * 1024,
        ),
    )(x, Wq, kpk, vpk, Wo)
